# baseline (speedup 1.0000x reference)
import functools

import jax
import jax.numpy as jnp
from jax import lax
from jax.experimental import pallas as pl
from jax.experimental.pallas import tpu as pltpu
from jax.experimental.pallas import tpu_sc as plsc

M, D, B = 100000, 64, 16384
NC, NS, L = 2, 16, 16
NW = NC * NS
RFULL = 3200
R_LAST = M - RFULL * (NW - 1)
CW = 384
VB = 128


def _mesh():
  return plsc.VectorSubcoreMesh(
      core_axis_name="c", subcore_axis_name="s", num_cores=NC, num_subcores=NS
  )


def _body(memT_hbm, cnt_hbm, valp_hbm, idx_hbm, outT_hbm, cntout_hbm,
          idx_v, mloc, mpos, last_pos, cnt_orig, cnt_new,
          sbuf, edgebuf, valbuf, colb, posb, a_b, b_b, sem1, sem2):
  wid = lax.axis_index("c") * NS + lax.axis_index("s")
  base = pl.multiple_of(wid * RFULL, 8)
  is_last = wid == (NW - 1)
  r_t = jnp.where(is_last, R_LAST, RFULL)
  iota = lax.broadcasted_iota(jnp.int32, (L,), 0)

  pltpu.sync_copy(idx_hbm, idx_v)

  @pl.when(jnp.logical_not(is_last))
  def _():
    pltpu.sync_copy(cnt_hbm.at[pl.ds(base, RFULL)], cnt_orig)
    pltpu.sync_copy(cnt_hbm.at[pl.ds(base, RFULL)], cnt_new)

  @pl.when(is_last)
  def _():
    pltpu.sync_copy(cnt_hbm.at[pl.ds(base, R_LAST)], cnt_orig.at[pl.ds(0, R_LAST)])
    pltpu.sync_copy(cnt_hbm.at[pl.ds(base, R_LAST)], cnt_new.at[pl.ds(0, R_LAST)])

  def scan_body(i, off):
    v = idx_v[pl.ds(i * L, L)]
    local = v - base
    m = (local >= 0) & (local < r_t)
    plsc.store_compressed(mloc.at[pl.ds(off, L)], local, mask=m)
    plsc.store_compressed(mpos.at[pl.ds(off, L)], iota + i * L, mask=m)
    return off + jnp.sum(m.astype(jnp.int32))

  k_n = lax.fori_loop(0, B // L, scan_body, jnp.int32(0))
  nmc = pl.cdiv(k_n, L)

  def dedup_body(j, _):
    lo = j * L
    vloc = mloc[pl.ds(lo, L)]
    vpos = mpos[pl.ds(lo, L)]
    valid = (iota + lo) < k_n
    later = jnp.zeros((L,), jnp.bool_)
    cnt_e = jnp.zeros((L,), jnp.int32)
    for s in range(1, L):
      v_dn = jnp.take_along_axis(vloc, jnp.minimum(iota + s, L - 1), axis=0)
      v_up = jnp.take_along_axis(vloc, jnp.maximum(iota - s, 0), axis=0)
      ok_dn = (iota + s < L) & ((lo + iota + s) < k_n)
      ok_up = iota - s >= 0
      later = later | (ok_dn & (v_dn == vloc))
      cnt_e = cnt_e + (ok_up & (v_up == vloc)).astype(jnp.int32)
    last = valid & jnp.logical_not(later)
    plsc.store_scatter(last_pos, [vloc], vpos, mask=last)
    plsc.addupdate_scatter(cnt_new, [vloc], cnt_e + 1, mask=last)
    return 0

  lax.fori_loop(0, nmc, dedup_body, 0)

  def win_body(j, woff):
    lo = j * L
    vloc = mloc[pl.ds(lo, L)]
    vpos = mpos[pl.ds(lo, L)]
    valid = (iota + lo) < k_n
    lp = plsc.load_gather(last_pos, [vloc], mask=valid)
    winner = valid & (lp == vpos)
    plsc.store_compressed(mloc.at[pl.ds(woff, L)], vloc, mask=winner)
    plsc.store_compressed(mpos.at[pl.ds(woff, L)], vpos, mask=winner)
    return woff + jnp.sum(winner.astype(jnp.int32))

  k_w = lax.fori_loop(0, nmc, win_body, jnp.int32(0))
  nwc = pl.cdiv(k_w, L)

  @pl.when(jnp.logical_not(is_last))
  def _():
    pltpu.sync_copy(cnt_new, cntout_hbm.at[pl.ds(base, RFULL)])

  @pl.when(is_last)
  def _():
    pltpu.sync_copy(cnt_new.at[pl.ds(0, R_LAST)], cntout_hbm.at[pl.ds(base, R_LAST)])

  def flush(fill, bufref):
    pos0 = posb[pl.ds(0, L)][0]

    def padp(g, _):
      sl = pl.ds(g * L, L)
      pv = posb[sl]
      posb[sl] = jnp.where(g * L + iota < fill, pv, pos0)
      return 0

    lax.fori_loop(0, VB // L, padp, 0)
    pltpu.async_copy(valp_hbm.at[posb.at[pl.ds(0, VB)]], valbuf, sem1).wait()

    def patch_group(g, _):
      sl16 = pl.ds(g * L, L)
      valid = (g * L + iota) < fill
      cvec = jnp.where(valid, colb[sl16], 0)
      avec = a_b[sl16]
      bvec = b_b[sl16]

      def patch_q(q, _):
        vs = [valbuf[g * L + j, pl.ds(q * L, L)] for j in range(L)]
        for s in (1, 2, 4, 8):
          dn = jnp.maximum(iota - s, 0)
          up = jnp.minimum(iota + s, L - 1)
          hi = (iota & s) == 0
          for i in range(L):
            if i & s:
              continue
            a, b = vs[i], vs[i + s]
            vs[i] = jnp.where(hi, a, jnp.take_along_axis(b, dn, axis=0))
            vs[i + s] = jnp.where(hi, jnp.take_along_axis(a, up, axis=0), b)
        for dd in range(L):
          d = q * L + dd
          dvec = jnp.broadcast_to(d, (L,))
          old = plsc.load_gather(bufref, [dvec, cvec], mask=valid)
          newv = avec * vs[dd] + bvec * old
          plsc.store_scatter(bufref, [dvec, cvec], newv, mask=valid)
        return 0

      lax.fori_loop(0, D // L, patch_q, 0)
      return 0

    lax.fori_loop(0, pl.cdiv(fill, L), patch_group, 0)

  def block_update(c0, cw, bufref):
    def chunk_body(j, fill):
      lo = j * L
      vloc = mloc[pl.ds(lo, L)]
      vpos = mpos[pl.ds(lo, L)]
      valid = (iota + lo) < k_w
      m = valid & (vloc + base >= c0) & (vloc + base < c0 + cw)
      plsc.store_compressed(colb.at[pl.ds(fill, L)], vloc + base - c0, mask=m)
      plsc.store_compressed(posb.at[pl.ds(fill, L)], vpos, mask=m)
      c = plsc.load_gather(cnt_orig, [vloc], mask=m)
      isnew = c == 0
      inv = 1.0 / jnp.where(isnew, 1, c).astype(jnp.float32)
      plsc.store_compressed(a_b.at[pl.ds(fill, L)],
                            jnp.where(isnew, 1.0, inv), mask=m)
      plsc.store_compressed(b_b.at[pl.ds(fill, L)],
                            jnp.where(isnew, 0.0, inv), mask=m)
      fill = fill + jnp.sum(m.astype(jnp.int32))

      def flush_branch():
        flush(fill, bufref)
        return jnp.int32(0)

      return jax.lax.cond(fill > VB - L, flush_branch, lambda: fill)

    fill = lax.fori_loop(0, nwc, chunk_body, jnp.int32(0))

    @pl.when(fill > 0)
    def _():
      flush(fill, bufref)

  nb = jnp.where(is_last, 2, RFULL // CW + 1)
  lim = jnp.where(is_last, M - (M % 128) - CW, base + RFULL - CW)

  def full_block(bi, _):
    c0 = pl.multiple_of(jnp.minimum(base + bi * CW, lim), 128)
    pltpu.sync_copy(memT_hbm.at[:, pl.ds(c0, CW)], sbuf)
    block_update(c0, CW, sbuf)
    pltpu.sync_copy(sbuf, outT_hbm.at[:, pl.ds(c0, CW)])
    return 0

  lax.fori_loop(0, nb, full_block, 0)

  @pl.when(is_last)
  def _():
    c0 = M - (M % 128)
    pltpu.sync_copy(memT_hbm.at[:, pl.ds(c0, M - c0)], edgebuf)
    block_update(c0, M - c0, edgebuf)
    pltpu.sync_copy(edgebuf, outT_hbm.at[:, pl.ds(c0, M - c0)])


@functools.cache
def _make_sc_store(interpret=False):
  return pl.kernel(
      _body,
      out_type=(
          jax.ShapeDtypeStruct((D, M), jnp.float32),
          jax.ShapeDtypeStruct((M,), jnp.int32),
      ),
      mesh=_mesh(),
      interpret=interpret,
      compiler_params=pltpu.CompilerParams(needs_layout_passes=False),
      scratch_types=[
          pltpu.VMEM((B,), jnp.int32),
          pltpu.VMEM((B,), jnp.int32),
          pltpu.VMEM((B,), jnp.int32),
          pltpu.VMEM((RFULL,), jnp.int32),
          pltpu.VMEM((RFULL,), jnp.int32),
          pltpu.VMEM((RFULL,), jnp.int32),
          pltpu.VMEM((D, CW), jnp.float32),
          pltpu.VMEM((D, 32), jnp.float32),
          pltpu.VMEM((VB, 2 * D), jnp.float32),
          pltpu.VMEM((VB + L,), jnp.int32),
          pltpu.VMEM((VB + L,), jnp.int32),
          pltpu.VMEM((VB + L,), jnp.float32),
          pltpu.VMEM((VB + L,), jnp.float32),
          pltpu.SemaphoreType.DMA,
          pltpu.SemaphoreType.DMA,
      ],
  )


def kernel(mem, counts, val, idx):
  memT = mem.T
  valp = jnp.pad(val, ((0, 0), (0, D)))
  outT, new_counts = _make_sc_store()(memT, counts, valp, idx)
  return outT.T, new_counts


# speedup vs baseline: 1.0664x; 1.0664x over previous
import functools

import jax
import jax.numpy as jnp
from jax import lax
from jax.experimental import pallas as pl
from jax.experimental.pallas import tpu as pltpu
from jax.experimental.pallas import tpu_sc as plsc

M, D, B = 100000, 64, 16384
NC, NS, L = 2, 16, 16
NW = NC * NS
RFULL = 3200
R_LAST = M - RFULL * (NW - 1)
CW = 384
VB = 128


def _mesh():
  return plsc.VectorSubcoreMesh(
      core_axis_name="c", subcore_axis_name="s", num_cores=NC, num_subcores=NS
  )


def _body(memT_hbm, cnt_hbm, valp_hbm, idx_hbm, outT_hbm, cntout_hbm,
          idx_v, mloc, mpos, last_pos, cnt_orig, cnt_new,
          sbuf, edgebuf, valbuf, colb, posb, a_b, b_b, sem1, sem2):
  wid = lax.axis_index("c") * NS + lax.axis_index("s")
  base = pl.multiple_of(wid * RFULL, 8)
  is_last = wid == (NW - 1)
  r_t = jnp.where(is_last, R_LAST, RFULL)
  iota = lax.broadcasted_iota(jnp.int32, (L,), 0)

  pltpu.sync_copy(idx_hbm, idx_v)

  @pl.when(jnp.logical_not(is_last))
  def _():
    pltpu.sync_copy(cnt_hbm.at[pl.ds(base, RFULL)], cnt_orig)
    pltpu.sync_copy(cnt_hbm.at[pl.ds(base, RFULL)], cnt_new)

  @pl.when(is_last)
  def _():
    pltpu.sync_copy(cnt_hbm.at[pl.ds(base, R_LAST)], cnt_orig.at[pl.ds(0, R_LAST)])
    pltpu.sync_copy(cnt_hbm.at[pl.ds(base, R_LAST)], cnt_new.at[pl.ds(0, R_LAST)])

  def scan_body(i, off):
    v = idx_v[pl.ds(i * L, L)]
    local = v - base
    m = (local >= 0) & (local < r_t)
    plsc.store_compressed(mloc.at[pl.ds(off, L)], local, mask=m)
    plsc.store_compressed(mpos.at[pl.ds(off, L)], iota + i * L, mask=m)
    return off + jnp.sum(m.astype(jnp.int32))

  k_n = lax.fori_loop(0, B // L, scan_body, jnp.int32(0))
  nmc = pl.cdiv(k_n, L)

  def dedup_body(j, _):
    lo = j * L
    vloc = mloc[pl.ds(lo, L)]
    vpos = mpos[pl.ds(lo, L)]
    valid = (iota + lo) < k_n
    later = jnp.zeros((L,), jnp.bool_)
    cnt_e = jnp.zeros((L,), jnp.int32)
    for s in range(1, L):
      v_dn = jnp.take_along_axis(vloc, jnp.minimum(iota + s, L - 1), axis=0)
      v_up = jnp.take_along_axis(vloc, jnp.maximum(iota - s, 0), axis=0)
      ok_dn = (iota + s < L) & ((lo + iota + s) < k_n)
      ok_up = iota - s >= 0
      later = later | (ok_dn & (v_dn == vloc))
      cnt_e = cnt_e + (ok_up & (v_up == vloc)).astype(jnp.int32)
    last = valid & jnp.logical_not(later)
    plsc.store_scatter(last_pos, [vloc], vpos, mask=last)
    plsc.addupdate_scatter(cnt_new, [vloc], cnt_e + 1, mask=last)
    return 0

  lax.fori_loop(0, nmc, dedup_body, 0)

  def win_body(j, woff):
    lo = j * L
    vloc = mloc[pl.ds(lo, L)]
    vpos = mpos[pl.ds(lo, L)]
    valid = (iota + lo) < k_n
    lp = plsc.load_gather(last_pos, [vloc], mask=valid)
    winner = valid & (lp == vpos)
    plsc.store_compressed(mloc.at[pl.ds(woff, L)], vloc, mask=winner)
    plsc.store_compressed(mpos.at[pl.ds(woff, L)], vpos, mask=winner)
    return woff + jnp.sum(winner.astype(jnp.int32))

  k_w = lax.fori_loop(0, nmc, win_body, jnp.int32(0))
  nwc = pl.cdiv(k_w, L)

  @pl.when(jnp.logical_not(is_last))
  def _():
    pltpu.sync_copy(cnt_new, cntout_hbm.at[pl.ds(base, RFULL)])

  @pl.when(is_last)
  def _():
    pltpu.sync_copy(cnt_new.at[pl.ds(0, R_LAST)], cntout_hbm.at[pl.ds(base, R_LAST)])

  def flush(fill, bufref):
    pos0 = posb[pl.ds(0, L)][0]

    def padp(g, _):
      sl = pl.ds(g * L, L)
      pv = posb[sl]
      posb[sl] = jnp.where(g * L + iota < fill, pv, pos0)
      return 0

    lax.fori_loop(0, VB // L, padp, 0)
    pltpu.async_copy(valp_hbm.at[posb.at[pl.ds(0, VB)]], valbuf, sem1).wait()

    def patch_group(g, _):
      sl16 = pl.ds(g * L, L)
      valid = (g * L + iota) < fill
      cvec = jnp.where(valid, colb[sl16], 0)
      avec = a_b[sl16]
      bvec = b_b[sl16]

      def patch_q(q, _):
        vs = [valbuf[g * L + j, pl.ds(q * L, L)] for j in range(L)]
        for s in (1, 2, 4, 8):
          dn = jnp.maximum(iota - s, 0)
          up = jnp.minimum(iota + s, L - 1)
          hi = (iota & s) == 0
          for i in range(L):
            if i & s:
              continue
            a, b = vs[i], vs[i + s]
            vs[i] = jnp.where(hi, a, jnp.take_along_axis(b, dn, axis=0))
            vs[i + s] = jnp.where(hi, jnp.take_along_axis(a, up, axis=0), b)
        for dd in range(L):
          d = q * L + dd
          dvec = jnp.broadcast_to(d, (L,))
          old = plsc.load_gather(bufref, [dvec, cvec], mask=valid)
          newv = avec * vs[dd] + bvec * old
          plsc.store_scatter(bufref, [dvec, cvec], newv, mask=valid)
        return 0

      lax.fori_loop(0, D // L, patch_q, 0)
      return 0

    lax.fori_loop(0, pl.cdiv(fill, L) * 0, patch_group, 0)

  def block_update(c0, cw, bufref):
    def chunk_body(j, fill):
      lo = j * L
      vloc = mloc[pl.ds(lo, L)]
      vpos = mpos[pl.ds(lo, L)]
      valid = (iota + lo) < k_w
      m = valid & (vloc + base >= c0) & (vloc + base < c0 + cw)
      plsc.store_compressed(colb.at[pl.ds(fill, L)], vloc + base - c0, mask=m)
      plsc.store_compressed(posb.at[pl.ds(fill, L)], vpos, mask=m)
      c = plsc.load_gather(cnt_orig, [vloc], mask=m)
      isnew = c == 0
      inv = 1.0 / jnp.where(isnew, 1, c).astype(jnp.float32)
      plsc.store_compressed(a_b.at[pl.ds(fill, L)],
                            jnp.where(isnew, 1.0, inv), mask=m)
      plsc.store_compressed(b_b.at[pl.ds(fill, L)],
                            jnp.where(isnew, 0.0, inv), mask=m)
      fill = fill + jnp.sum(m.astype(jnp.int32))

      def flush_branch():
        flush(fill, bufref)
        return jnp.int32(0)

      return jax.lax.cond(fill > VB - L, flush_branch, lambda: fill)

    fill = lax.fori_loop(0, nwc, chunk_body, jnp.int32(0))

    @pl.when(fill > 0)
    def _():
      flush(fill, bufref)

  nb = jnp.where(is_last, 2, RFULL // CW + 1)
  lim = jnp.where(is_last, M - (M % 128) - CW, base + RFULL - CW)

  def full_block(bi, _):
    c0 = pl.multiple_of(jnp.minimum(base + bi * CW, lim), 128)
    pltpu.sync_copy(memT_hbm.at[:, pl.ds(c0, CW)], sbuf)
    block_update(c0, CW, sbuf)
    pltpu.sync_copy(sbuf, outT_hbm.at[:, pl.ds(c0, CW)])
    return 0

  lax.fori_loop(0, nb, full_block, 0)

  @pl.when(is_last)
  def _():
    c0 = M - (M % 128)
    pltpu.sync_copy(memT_hbm.at[:, pl.ds(c0, M - c0)], edgebuf)
    block_update(c0, M - c0, edgebuf)
    pltpu.sync_copy(edgebuf, outT_hbm.at[:, pl.ds(c0, M - c0)])


@functools.cache
def _make_sc_store(interpret=False):
  return pl.kernel(
      _body,
      out_type=(
          jax.ShapeDtypeStruct((D, M), jnp.float32),
          jax.ShapeDtypeStruct((M,), jnp.int32),
      ),
      mesh=_mesh(),
      interpret=interpret,
      compiler_params=pltpu.CompilerParams(needs_layout_passes=False),
      scratch_types=[
          pltpu.VMEM((B,), jnp.int32),
          pltpu.VMEM((B,), jnp.int32),
          pltpu.VMEM((B,), jnp.int32),
          pltpu.VMEM((RFULL,), jnp.int32),
          pltpu.VMEM((RFULL,), jnp.int32),
          pltpu.VMEM((RFULL,), jnp.int32),
          pltpu.VMEM((D, CW), jnp.float32),
          pltpu.VMEM((D, 32), jnp.float32),
          pltpu.VMEM((VB, 2 * D), jnp.float32),
          pltpu.VMEM((VB + L,), jnp.int32),
          pltpu.VMEM((VB + L,), jnp.int32),
          pltpu.VMEM((VB + L,), jnp.float32),
          pltpu.VMEM((VB + L,), jnp.float32),
          pltpu.SemaphoreType.DMA,
          pltpu.SemaphoreType.DMA,
      ],
  )


def kernel(mem, counts, val, idx):
  memT = mem.T
  valp = jnp.pad(val, ((0, 0), (0, D)))
  outT, new_counts = _make_sc_store()(memT, counts, valp, idx)
  return outT.T, new_counts


# speedup vs baseline: 1.0780x; 1.0109x over previous
import functools

import jax
import jax.numpy as jnp
from jax import lax
from jax.experimental import pallas as pl
from jax.experimental.pallas import tpu as pltpu
from jax.experimental.pallas import tpu_sc as plsc

M, D, B = 100000, 64, 16384
NC, NS, L = 2, 16, 16
NW = NC * NS
RFULL = 3200
R_LAST = M - RFULL * (NW - 1)
CW = 384
VB = 112


def _mesh():
  return plsc.VectorSubcoreMesh(
      core_axis_name="c", subcore_axis_name="s", num_cores=NC, num_subcores=NS
  )


def _body(memT_hbm, cnt_hbm, valp_hbm, idx_hbm, outT_hbm, cntout_hbm,
          idx_v, mloc, mpos, ab_all, last_pos, cnt_orig, cnt_new,
          sbuf, edgebuf, valbuf, colb, posb, ab_b, sem1, sem2):
  wid = lax.axis_index("c") * NS + lax.axis_index("s")
  base = pl.multiple_of(wid * RFULL, 8)
  is_last = wid == (NW - 1)
  r_t = jnp.where(is_last, R_LAST, RFULL)
  iota = lax.broadcasted_iota(jnp.int32, (L,), 0)

  pltpu.sync_copy(idx_hbm, idx_v)

  @pl.when(jnp.logical_not(is_last))
  def _():
    pltpu.sync_copy(cnt_hbm.at[pl.ds(base, RFULL)], cnt_orig)
    pltpu.sync_copy(cnt_hbm.at[pl.ds(base, RFULL)], cnt_new)

  @pl.when(is_last)
  def _():
    pltpu.sync_copy(cnt_hbm.at[pl.ds(base, R_LAST)], cnt_orig.at[pl.ds(0, R_LAST)])
    pltpu.sync_copy(cnt_hbm.at[pl.ds(base, R_LAST)], cnt_new.at[pl.ds(0, R_LAST)])

  def scan_body(i, off):
    v = idx_v[pl.ds(i * L, L)]
    local = v - base
    m = (local >= 0) & (local < r_t)
    plsc.store_compressed(mloc.at[pl.ds(off, L)], local, mask=m)
    plsc.store_compressed(mpos.at[pl.ds(off, L)], iota + i * L, mask=m)
    return off + jnp.sum(m.astype(jnp.int32))

  k_n = lax.fori_loop(0, B // L, scan_body, jnp.int32(0))
  nmc = pl.cdiv(k_n, L)

  def dedup_body(j, _):
    lo = j * L
    vloc = mloc[pl.ds(lo, L)]
    vpos = mpos[pl.ds(lo, L)]
    valid = (iota + lo) < k_n
    later = jnp.zeros((L,), jnp.bool_)
    cnt_e = jnp.zeros((L,), jnp.int32)
    for s in range(1, L):
      v_dn = jnp.take_along_axis(vloc, jnp.minimum(iota + s, L - 1), axis=0)
      v_up = jnp.take_along_axis(vloc, jnp.maximum(iota - s, 0), axis=0)
      ok_dn = (iota + s < L) & ((lo + iota + s) < k_n)
      ok_up = iota - s >= 0
      later = later | (ok_dn & (v_dn == vloc))
      cnt_e = cnt_e + (ok_up & (v_up == vloc)).astype(jnp.int32)
    last = valid & jnp.logical_not(later)
    plsc.store_scatter(last_pos, [vloc], vpos, mask=last)
    plsc.addupdate_scatter(cnt_new, [vloc], cnt_e + 1, mask=last)
    return 0

  lax.fori_loop(0, nmc, dedup_body, 0)

  def win_body(j, woff):
    lo = j * L
    vloc = mloc[pl.ds(lo, L)]
    vpos = mpos[pl.ds(lo, L)]
    valid = (iota + lo) < k_n
    lp = plsc.load_gather(last_pos, [vloc], mask=valid)
    winner = valid & (lp == vpos)
    plsc.store_compressed(mloc.at[pl.ds(woff, L)], vloc, mask=winner)
    plsc.store_compressed(mpos.at[pl.ds(woff, L)], vpos, mask=winner)
    c = plsc.load_gather(cnt_orig, [vloc], mask=winner)
    isnew = c == 0
    inv = 1.0 / jnp.where(isnew, 1, c).astype(jnp.float32)
    plsc.store_compressed(ab_all.at[pl.ds(woff, L)],
                          jnp.where(isnew, -1.0, inv), mask=winner)
    return woff + jnp.sum(winner.astype(jnp.int32))

  k_w = lax.fori_loop(0, nmc, win_body, jnp.int32(0))
  nwc = pl.cdiv(k_w, L)

  @pl.when(jnp.logical_not(is_last))
  def _():
    pltpu.sync_copy(cnt_new, cntout_hbm.at[pl.ds(base, RFULL)])

  @pl.when(is_last)
  def _():
    pltpu.sync_copy(cnt_new.at[pl.ds(0, R_LAST)], cntout_hbm.at[pl.ds(base, R_LAST)])

  def flush(fill, bufref):
    pos0 = posb[pl.ds(0, L)][0]

    def padp(g, _):
      sl = pl.ds(g * L, L)
      pv = posb[sl]
      posb[sl] = jnp.where(g * L + iota < fill, pv, pos0)
      return 0

    lax.fori_loop(0, VB // L, padp, 0)
    pltpu.async_copy(valp_hbm.at[posb.at[pl.ds(0, VB)]], valbuf, sem1).wait()

    def patch_group(g, _):
      sl16 = pl.ds(g * L, L)
      valid = (g * L + iota) < fill
      cvec = jnp.where(valid, colb[sl16], 0)
      ab = ab_b[sl16]
      isnew = ab < 0
      avec = jnp.where(isnew, 1.0, ab)
      bvec = jnp.where(isnew, 0.0, ab)

      def patch_q(q, _):
        vs = [valbuf[g * L + j, pl.ds(q * L, L)] for j in range(L)]
        for s in (1, 2, 4, 8):
          dn = jnp.maximum(iota - s, 0)
          up = jnp.minimum(iota + s, L - 1)
          hi = (iota & s) == 0
          for i in range(L):
            if i & s:
              continue
            a, b = vs[i], vs[i + s]
            vs[i] = jnp.where(hi, a, jnp.take_along_axis(b, dn, axis=0))
            vs[i + s] = jnp.where(hi, jnp.take_along_axis(a, up, axis=0), b)
        for dd in range(L):
          d = q * L + dd
          dvec = jnp.broadcast_to(d, (L,))
          old = plsc.load_gather(bufref, [dvec, cvec], mask=valid)
          newv = avec * vs[dd] + bvec * old
          plsc.store_scatter(bufref, [dvec, cvec], newv, mask=valid)
        return 0

      lax.fori_loop(0, D // L, patch_q, 0)
      return 0

    lax.fori_loop(0, pl.cdiv(fill, L), patch_group, 0)

  def block_update(c0, cw, bufref):
    def chunk_body(j, fill):
      lo = j * L
      vloc = mloc[pl.ds(lo, L)]
      vpos = mpos[pl.ds(lo, L)]
      valid = (iota + lo) < k_w
      m = valid & (vloc + base >= c0) & (vloc + base < c0 + cw)
      plsc.store_compressed(colb.at[pl.ds(fill, L)], vloc + base - c0, mask=m)
      plsc.store_compressed(posb.at[pl.ds(fill, L)], vpos, mask=m)
      plsc.store_compressed(ab_b.at[pl.ds(fill, L)], ab_all[pl.ds(lo, L)],
                            mask=m)
      fill = fill + jnp.sum(m.astype(jnp.int32))

      def flush_branch():
        flush(fill, bufref)
        return jnp.int32(0)

      return jax.lax.cond(fill > VB - L, flush_branch, lambda: fill)

    fill = lax.fori_loop(0, nwc, chunk_body, jnp.int32(0))

    @pl.when(fill > 0)
    def _():
      flush(fill, bufref)

  nb = jnp.where(is_last, 2, RFULL // CW + 1)
  lim = jnp.where(is_last, M - (M % 128) - CW, base + RFULL - CW)

  def full_block(bi, _):
    c0 = pl.multiple_of(jnp.minimum(base + bi * CW, lim), 128)
    pltpu.sync_copy(memT_hbm.at[:, pl.ds(c0, CW)], sbuf)
    block_update(c0, CW, sbuf)
    pltpu.sync_copy(sbuf, outT_hbm.at[:, pl.ds(c0, CW)])
    return 0

  lax.fori_loop(0, nb, full_block, 0)

  @pl.when(is_last)
  def _():
    c0 = M - (M % 128)
    pltpu.sync_copy(memT_hbm.at[:, pl.ds(c0, M - c0)], edgebuf)
    block_update(c0, M - c0, edgebuf)
    pltpu.sync_copy(edgebuf, outT_hbm.at[:, pl.ds(c0, M - c0)])


@functools.cache
def _make_sc_store(interpret=False):
  return pl.kernel(
      _body,
      out_type=(
          jax.ShapeDtypeStruct((D, M), jnp.float32),
          jax.ShapeDtypeStruct((M,), jnp.int32),
      ),
      mesh=_mesh(),
      interpret=interpret,
      compiler_params=pltpu.CompilerParams(needs_layout_passes=False),
      scratch_types=[
          pltpu.VMEM((B,), jnp.int32),
          pltpu.VMEM((B,), jnp.int32),
          pltpu.VMEM((B,), jnp.int32),
          pltpu.VMEM((B,), jnp.float32),
          pltpu.VMEM((RFULL,), jnp.int32),
          pltpu.VMEM((RFULL,), jnp.int32),
          pltpu.VMEM((RFULL,), jnp.int32),
          pltpu.VMEM((D, CW), jnp.float32),
          pltpu.VMEM((D, 32), jnp.float32),
          pltpu.VMEM((VB, 2 * D), jnp.float32),
          pltpu.VMEM((VB + L,), jnp.int32),
          pltpu.VMEM((VB + L,), jnp.int32),
          pltpu.VMEM((VB + L,), jnp.float32),
          pltpu.SemaphoreType.DMA,
          pltpu.SemaphoreType.DMA,
      ],
  )


def kernel(mem, counts, val, idx):
  memT = mem.T
  valp = jnp.pad(val, ((0, 0), (0, D)))
  outT, new_counts = _make_sc_store()(memT, counts, valp, idx)
  return outT.T, new_counts


# speedup vs baseline: 1.0782x; 1.0002x over previous
import functools

import jax
import jax.numpy as jnp
from jax import lax
from jax.experimental import pallas as pl
from jax.experimental.pallas import tpu as pltpu
from jax.experimental.pallas import tpu_sc as plsc

M, D, B = 100000, 64, 16384
NC, NS, L = 2, 16, 16
NW = NC * NS
RFULL = 3200
R_LAST = M - RFULL * (NW - 1)
CW = 384
VB = 112


def _mesh():
  return plsc.VectorSubcoreMesh(
      core_axis_name="c", subcore_axis_name="s", num_cores=NC, num_subcores=NS
  )


def _body(memT_hbm, cnt_hbm, valp_hbm, idx_hbm, outT_hbm, cntout_hbm,
          idx_v, mloc, mpos, ab_all, last_pos, cnt_orig, cnt_new,
          sbuf, edgebuf, valbuf, colb, posb, ab_b, sem1, sem2):
  wid = lax.axis_index("c") * NS + lax.axis_index("s")
  base = pl.multiple_of(wid * RFULL, 8)
  is_last = wid == (NW - 1)
  r_t = jnp.where(is_last, R_LAST, RFULL)
  iota = lax.broadcasted_iota(jnp.int32, (L,), 0)

  pltpu.sync_copy(idx_hbm, idx_v)

  @pl.when(jnp.logical_not(is_last))
  def _():
    pltpu.sync_copy(cnt_hbm.at[pl.ds(base, RFULL)], cnt_orig)
    pltpu.sync_copy(cnt_hbm.at[pl.ds(base, RFULL)], cnt_new)

  @pl.when(is_last)
  def _():
    pltpu.sync_copy(cnt_hbm.at[pl.ds(base, R_LAST)], cnt_orig.at[pl.ds(0, R_LAST)])
    pltpu.sync_copy(cnt_hbm.at[pl.ds(base, R_LAST)], cnt_new.at[pl.ds(0, R_LAST)])

  def scan_body(i, off):
    v = idx_v[pl.ds(i * L, L)]
    local = v - base
    m = (local >= 0) & (local < r_t)
    plsc.store_compressed(mloc.at[pl.ds(off, L)], local, mask=m)
    plsc.store_compressed(mpos.at[pl.ds(off, L)], iota + i * L, mask=m)
    return off + jnp.sum(m.astype(jnp.int32))

  with jax.named_scope("p1_scan"):
    k_n = lax.fori_loop(0, B // L, scan_body, jnp.int32(0))
  nmc = pl.cdiv(k_n, L)

  def dedup_body(j, _):
    lo = j * L
    vloc = mloc[pl.ds(lo, L)]
    vpos = mpos[pl.ds(lo, L)]
    valid = (iota + lo) < k_n
    later = jnp.zeros((L,), jnp.bool_)
    cnt_e = jnp.zeros((L,), jnp.int32)
    for s in range(1, L):
      v_dn = jnp.take_along_axis(vloc, jnp.minimum(iota + s, L - 1), axis=0)
      v_up = jnp.take_along_axis(vloc, jnp.maximum(iota - s, 0), axis=0)
      ok_dn = (iota + s < L) & ((lo + iota + s) < k_n)
      ok_up = iota - s >= 0
      later = later | (ok_dn & (v_dn == vloc))
      cnt_e = cnt_e + (ok_up & (v_up == vloc)).astype(jnp.int32)
    last = valid & jnp.logical_not(later)
    plsc.store_scatter(last_pos, [vloc], vpos, mask=last)
    plsc.addupdate_scatter(cnt_new, [vloc], cnt_e + 1, mask=last)
    return 0

  with jax.named_scope("p2_dedup"):
    lax.fori_loop(0, nmc, dedup_body, 0)

  def win_body(j, woff):
    lo = j * L
    vloc = mloc[pl.ds(lo, L)]
    vpos = mpos[pl.ds(lo, L)]
    valid = (iota + lo) < k_n
    lp = plsc.load_gather(last_pos, [vloc], mask=valid)
    winner = valid & (lp == vpos)
    plsc.store_compressed(mloc.at[pl.ds(woff, L)], vloc, mask=winner)
    plsc.store_compressed(mpos.at[pl.ds(woff, L)], vpos, mask=winner)
    c = plsc.load_gather(cnt_orig, [vloc], mask=winner)
    isnew = c == 0
    inv = 1.0 / jnp.where(isnew, 1, c).astype(jnp.float32)
    plsc.store_compressed(ab_all.at[pl.ds(woff, L)],
                          jnp.where(isnew, -1.0, inv), mask=winner)
    return woff + jnp.sum(winner.astype(jnp.int32))

  with jax.named_scope("p3_winners"):
    k_w = lax.fori_loop(0, nmc, win_body, jnp.int32(0))
  nwc = pl.cdiv(k_w, L)

  @pl.when(jnp.logical_not(is_last))
  def _():
    pltpu.sync_copy(cnt_new, cntout_hbm.at[pl.ds(base, RFULL)])

  @pl.when(is_last)
  def _():
    pltpu.sync_copy(cnt_new.at[pl.ds(0, R_LAST)], cntout_hbm.at[pl.ds(base, R_LAST)])

  def flush(fill, bufref):
    pos0 = posb[pl.ds(0, L)][0]

    def padp(g, _):
      sl = pl.ds(g * L, L)
      pv = posb[sl]
      posb[sl] = jnp.where(g * L + iota < fill, pv, pos0)
      return 0

    lax.fori_loop(0, VB // L, padp, 0)
    pltpu.async_copy(valp_hbm.at[posb.at[pl.ds(0, VB)]], valbuf, sem1).wait()

    def patch_group(g, _):
      sl16 = pl.ds(g * L, L)
      valid = (g * L + iota) < fill
      cvec = jnp.where(valid, colb[sl16], 0)
      ab = ab_b[sl16]
      isnew = ab < 0
      avec = jnp.where(isnew, 1.0, ab)
      bvec = jnp.where(isnew, 0.0, ab)

      def patch_q(q, _):
        vs = [valbuf[g * L + j, pl.ds(q * L, L)] for j in range(L)]
        for s in (1, 2, 4, 8):
          dn = jnp.maximum(iota - s, 0)
          up = jnp.minimum(iota + s, L - 1)
          hi = (iota & s) == 0
          for i in range(L):
            if i & s:
              continue
            a, b = vs[i], vs[i + s]
            vs[i] = jnp.where(hi, a, jnp.take_along_axis(b, dn, axis=0))
            vs[i + s] = jnp.where(hi, jnp.take_along_axis(a, up, axis=0), b)
        for dd in range(L):
          d = q * L + dd
          dvec = jnp.broadcast_to(d, (L,))
          old = plsc.load_gather(bufref, [dvec, cvec], mask=valid)
          newv = avec * vs[dd] + bvec * old
          plsc.store_scatter(bufref, [dvec, cvec], newv, mask=valid)
        return 0

      lax.fori_loop(0, D // L, patch_q, 0)
      return 0

    lax.fori_loop(0, pl.cdiv(fill, L), patch_group, 0)

  def block_update(c0, cw, bufref):
    def chunk_body(j, fill):
      lo = j * L
      vloc = mloc[pl.ds(lo, L)]
      vpos = mpos[pl.ds(lo, L)]
      valid = (iota + lo) < k_w
      m = valid & (vloc + base >= c0) & (vloc + base < c0 + cw)
      plsc.store_compressed(colb.at[pl.ds(fill, L)], vloc + base - c0, mask=m)
      plsc.store_compressed(posb.at[pl.ds(fill, L)], vpos, mask=m)
      plsc.store_compressed(ab_b.at[pl.ds(fill, L)], ab_all[pl.ds(lo, L)],
                            mask=m)
      fill = fill + jnp.sum(m.astype(jnp.int32))

      def flush_branch():
        flush(fill, bufref)
        return jnp.int32(0)

      return jax.lax.cond(fill > VB - L, flush_branch, lambda: fill)

    fill = lax.fori_loop(0, nwc, chunk_body, jnp.int32(0))

    @pl.when(fill > 0)
    def _():
      flush(fill, bufref)

  nb = jnp.where(is_last, 2, RFULL // CW + 1)
  lim = jnp.where(is_last, M - (M % 128) - CW, base + RFULL - CW)

  def full_block(bi, _):
    c0 = pl.multiple_of(jnp.minimum(base + bi * CW, lim), 128)
    with jax.named_scope("s_in"):
      pltpu.sync_copy(memT_hbm.at[:, pl.ds(c0, CW)], sbuf)
    with jax.named_scope("s_upd"):
      block_update(c0, CW, sbuf)
    with jax.named_scope("s_out"):
      pltpu.sync_copy(sbuf, outT_hbm.at[:, pl.ds(c0, CW)])
    return 0

  with jax.named_scope("p4_stream"):
    lax.fori_loop(0, nb, full_block, 0)

  @pl.when(is_last)
  def _():
    c0 = M - (M % 128)
    pltpu.sync_copy(memT_hbm.at[:, pl.ds(c0, M - c0)], edgebuf)
    block_update(c0, M - c0, edgebuf)
    pltpu.sync_copy(edgebuf, outT_hbm.at[:, pl.ds(c0, M - c0)])


@functools.cache
def _make_sc_store(interpret=False):
  return pl.kernel(
      _body,
      out_type=(
          jax.ShapeDtypeStruct((D, M), jnp.float32),
          jax.ShapeDtypeStruct((M,), jnp.int32),
      ),
      mesh=_mesh(),
      interpret=interpret,
      compiler_params=pltpu.CompilerParams(needs_layout_passes=False),
      scratch_types=[
          pltpu.VMEM((B,), jnp.int32),
          pltpu.VMEM((B,), jnp.int32),
          pltpu.VMEM((B,), jnp.int32),
          pltpu.VMEM((B,), jnp.float32),
          pltpu.VMEM((RFULL,), jnp.int32),
          pltpu.VMEM((RFULL,), jnp.int32),
          pltpu.VMEM((RFULL,), jnp.int32),
          pltpu.VMEM((D, CW), jnp.float32),
          pltpu.VMEM((D, 32), jnp.float32),
          pltpu.VMEM((VB, 2 * D), jnp.float32),
          pltpu.VMEM((VB + L,), jnp.int32),
          pltpu.VMEM((VB + L,), jnp.int32),
          pltpu.VMEM((VB + L,), jnp.float32),
          pltpu.SemaphoreType.DMA,
          pltpu.SemaphoreType.DMA,
      ],
  )


def kernel(mem, counts, val, idx):
  memT = mem.T
  valp = jnp.pad(val, ((0, 0), (0, D)))
  outT, new_counts = _make_sc_store()(memT, counts, valp, idx)
  return outT.T, new_counts


# speedup vs baseline: 1.1456x; 1.0625x over previous
import functools

import jax
import jax.numpy as jnp
from jax import lax
from jax.experimental import pallas as pl
from jax.experimental.pallas import tpu as pltpu
from jax.experimental.pallas import tpu_sc as plsc

M, D, B = 100000, 64, 16384
NC, NS, L = 2, 16, 16
NW = NC * NS
RFULL = 3200
R_LAST = M - RFULL * (NW - 1)
CW = 384
VB = 112
LCAP = -(-CW // VB) * VB + L


def _mesh():
  return plsc.VectorSubcoreMesh(
      core_axis_name="c", subcore_axis_name="s", num_cores=NC, num_subcores=NS
  )


def _body(memT_hbm, cnt_hbm, valp_hbm, idx_hbm, outT_hbm, cntout_hbm,
          idx_v, mloc, mpos, last_pos, cnt_orig, cnt_new,
          sbuf, edgebuf, valbuf, colb, posb, ab_b, sem1, sem2):
  wid = lax.axis_index("c") * NS + lax.axis_index("s")
  base = pl.multiple_of(wid * RFULL, 8)
  is_last = wid == (NW - 1)
  r_t = jnp.where(is_last, R_LAST, RFULL)
  iota = lax.broadcasted_iota(jnp.int32, (L,), 0)

  pltpu.sync_copy(idx_hbm, idx_v)

  @pl.when(jnp.logical_not(is_last))
  def _():
    pltpu.sync_copy(cnt_hbm.at[pl.ds(base, RFULL)], cnt_orig)
    pltpu.sync_copy(cnt_hbm.at[pl.ds(base, RFULL)], cnt_new)

  @pl.when(is_last)
  def _():
    pltpu.sync_copy(cnt_hbm.at[pl.ds(base, R_LAST)], cnt_orig.at[pl.ds(0, R_LAST)])
    pltpu.sync_copy(cnt_hbm.at[pl.ds(base, R_LAST)], cnt_new.at[pl.ds(0, R_LAST)])

  def scan_body(i, off):
    v = idx_v[pl.ds(i * L, L)]
    local = v - base
    m = (local >= 0) & (local < r_t)
    plsc.store_compressed(mloc.at[pl.ds(off, L)], local, mask=m)
    plsc.store_compressed(mpos.at[pl.ds(off, L)], iota + i * L, mask=m)
    return off + jnp.sum(m.astype(jnp.int32))

  with jax.named_scope("p1_scan"):
    k_n = lax.fori_loop(0, B // L, scan_body, jnp.int32(0))
  nmc = pl.cdiv(k_n, L)

  def dedup_body(j, _):
    lo = j * L
    vloc = mloc[pl.ds(lo, L)]
    vpos = mpos[pl.ds(lo, L)]
    valid = (iota + lo) < k_n
    later = jnp.zeros((L,), jnp.bool_)
    cnt_e = jnp.zeros((L,), jnp.int32)
    for s in range(1, L):
      v_dn = jnp.take_along_axis(vloc, jnp.minimum(iota + s, L - 1), axis=0)
      v_up = jnp.take_along_axis(vloc, jnp.maximum(iota - s, 0), axis=0)
      ok_dn = (iota + s < L) & ((lo + iota + s) < k_n)
      ok_up = iota - s >= 0
      later = later | (ok_dn & (v_dn == vloc))
      cnt_e = cnt_e + (ok_up & (v_up == vloc)).astype(jnp.int32)
    last = valid & jnp.logical_not(later)
    plsc.store_scatter(last_pos, [vloc], vpos, mask=last)
    plsc.addupdate_scatter(cnt_new, [vloc], cnt_e + 1, mask=last)
    return 0

  with jax.named_scope("p2_dedup"):
    lax.fori_loop(0, nmc, dedup_body, 0)

  def win_body(j, woff):
    lo = j * L
    vloc = mloc[pl.ds(lo, L)]
    vpos = mpos[pl.ds(lo, L)]
    valid = (iota + lo) < k_n
    lp = plsc.load_gather(last_pos, [vloc], mask=valid)
    winner = valid & (lp == vpos)
    plsc.store_compressed(mloc.at[pl.ds(woff, L)], vloc, mask=winner)
    plsc.store_compressed(mpos.at[pl.ds(woff, L)], vpos, mask=winner)
    return woff + jnp.sum(winner.astype(jnp.int32))

  with jax.named_scope("p3_winners"):
    k_w = lax.fori_loop(0, nmc, win_body, jnp.int32(0))
  nwc = pl.cdiv(k_w, L)

  @pl.when(jnp.logical_not(is_last))
  def _():
    pltpu.sync_copy(cnt_new, cntout_hbm.at[pl.ds(base, RFULL)])

  @pl.when(is_last)
  def _():
    pltpu.sync_copy(cnt_new.at[pl.ds(0, R_LAST)], cntout_hbm.at[pl.ds(base, R_LAST)])

  def rescan(c0, cw):
    def chunk_body(j, fill):
      lo = j * L
      vloc = mloc[pl.ds(lo, L)]
      vpos = mpos[pl.ds(lo, L)]
      valid = (iota + lo) < k_w
      m = valid & (vloc + base >= c0) & (vloc + base < c0 + cw)
      plsc.store_compressed(colb.at[pl.ds(fill, L)], vloc + base - c0, mask=m)
      plsc.store_compressed(posb.at[pl.ds(fill, L)], vpos, mask=m)
      c = plsc.load_gather(cnt_orig, [vloc], mask=m)
      isnew = c == 0
      inv = 1.0 / jnp.where(isnew, 1, c).astype(jnp.float32)
      plsc.store_compressed(ab_b.at[pl.ds(fill, L)],
                            jnp.where(isnew, -1.0, inv), mask=m)
      return fill + jnp.sum(m.astype(jnp.int32))

    fill = lax.fori_loop(0, nwc, chunk_body, jnp.int32(0))

    @pl.when(fill > 0)
    def _():
      pos0 = posb[pl.ds(0, L)][0]

      def padp(g, _):
        sl = pl.ds(g * L, L)
        posb[sl] = jnp.where(g * L + iota < fill, posb[sl], pos0)
        return 0

      lax.fori_loop(fill // L, pl.cdiv(fill, VB) * (VB // L), padp, 0)
    return fill

  def patch_all(fill, bufref):
    def batch_body(bat, _):
      pltpu.async_copy(valp_hbm.at[posb.at[pl.ds(bat * VB, VB)]],
                       valbuf, sem2).wait()

      def patch_group(gl, _):
        g = bat * (VB // L) + gl
        sl16 = pl.ds(g * L, L)
        valid = (g * L + iota) < fill
        cvec = jnp.where(valid, colb[sl16], 0)
        ab = ab_b[sl16]
        isnew = ab < 0
        avec = jnp.where(isnew, 1.0, ab)
        bvec = jnp.where(isnew, 0.0, ab)

        def patch_q(q, _):
          vs = [valbuf[gl * L + j, pl.ds(q * L, L)] for j in range(L)]
          for s in (1, 2, 4, 8):
            dn = jnp.maximum(iota - s, 0)
            up = jnp.minimum(iota + s, L - 1)
            hi = (iota & s) == 0
            for i in range(L):
              if i & s:
                continue
              a, b = vs[i], vs[i + s]
              vs[i] = jnp.where(hi, a, jnp.take_along_axis(b, dn, axis=0))
              vs[i + s] = jnp.where(hi, jnp.take_along_axis(a, up, axis=0), b)
          for dd in range(L):
            d = q * L + dd
            dvec = jnp.broadcast_to(d, (L,))
            old = plsc.load_gather(bufref, [dvec, cvec], mask=valid)
            newv = avec * vs[dd] + bvec * old
            plsc.store_scatter(bufref, [dvec, cvec], newv, mask=valid)
          return 0

        lax.fori_loop(0, D // L, patch_q, 0)
        return 0

      ngl = jnp.minimum(pl.cdiv(fill - bat * VB, L), VB // L)
      lax.fori_loop(0, ngl, patch_group, 0)
      return 0

    lax.fori_loop(0, pl.cdiv(fill, VB), batch_body, 0)

  nb = jnp.where(is_last, 2, RFULL // CW + 1)
  lim = jnp.where(is_last, M - (M % 128) - CW, base + RFULL - CW)

  def full_block(bi, _):
    c0 = pl.multiple_of(jnp.minimum(base + bi * CW, lim), 128)
    cp_in = pltpu.async_copy(memT_hbm.at[:, pl.ds(c0, CW)], sbuf, sem1)
    with jax.named_scope("s_scan"):
      fill = rescan(c0, CW)
    with jax.named_scope("s_in"):
      cp_in.wait()
    with jax.named_scope("s_upd"):
      patch_all(fill, sbuf)
    with jax.named_scope("s_out"):
      pltpu.sync_copy(sbuf, outT_hbm.at[:, pl.ds(c0, CW)])
    return 0

  with jax.named_scope("p4_stream"):
    lax.fori_loop(0, nb, full_block, 0)

  @pl.when(is_last)
  def _():
    c0 = M - (M % 128)
    pltpu.sync_copy(memT_hbm.at[:, pl.ds(c0, M - c0)], edgebuf)
    patch_all(rescan(c0, M - c0), edgebuf)
    pltpu.sync_copy(edgebuf, outT_hbm.at[:, pl.ds(c0, M - c0)])


@functools.cache
def _make_sc_store(interpret=False):
  return pl.kernel(
      _body,
      out_type=(
          jax.ShapeDtypeStruct((D, M), jnp.float32),
          jax.ShapeDtypeStruct((M,), jnp.int32),
      ),
      mesh=_mesh(),
      interpret=interpret,
      compiler_params=pltpu.CompilerParams(needs_layout_passes=False),
      scratch_types=[
          pltpu.VMEM((B,), jnp.int32),
          pltpu.VMEM((B,), jnp.int32),
          pltpu.VMEM((B,), jnp.int32),
          pltpu.VMEM((RFULL,), jnp.int32),
          pltpu.VMEM((RFULL,), jnp.int32),
          pltpu.VMEM((RFULL,), jnp.int32),
          pltpu.VMEM((D, CW), jnp.float32),
          pltpu.VMEM((D, 32), jnp.float32),
          pltpu.VMEM((VB, 2 * D), jnp.float32),
          pltpu.VMEM((LCAP,), jnp.int32),
          pltpu.VMEM((LCAP,), jnp.int32),
          pltpu.VMEM((LCAP,), jnp.float32),
          pltpu.SemaphoreType.DMA,
          pltpu.SemaphoreType.DMA,
      ],
  )


def kernel(mem, counts, val, idx):
  memT = mem.T
  valp = jnp.pad(val, ((0, 0), (0, D)))
  outT, new_counts = _make_sc_store()(memT, counts, valp, idx)
  return outT.T, new_counts


# speedup vs baseline: 1.1462x; 1.0005x over previous
import functools

import jax
import jax.numpy as jnp
from jax import lax
from jax.experimental import pallas as pl
from jax.experimental.pallas import tpu as pltpu
from jax.experimental.pallas import tpu_sc as plsc

M, D, B = 100000, 64, 16384
NC, NS, L = 2, 16, 16
NW = NC * NS
RFULL = 3200
R_LAST = M - RFULL * (NW - 1)
CW = 384
VB = 112
LCAP = -(-CW // VB) * VB + L


def _mesh():
  return plsc.VectorSubcoreMesh(
      core_axis_name="c", subcore_axis_name="s", num_cores=NC, num_subcores=NS
  )


def _body(memT_hbm, cnt_hbm, valp_hbm, idx_hbm, outT_hbm, cntout_hbm,
          idx_v, mloc, mpos, last_pos, cnt_orig, cnt_new,
          sbuf, edgebuf, valbuf, colb, posb, ab_b, sem1, sem2):
  wid = lax.axis_index("c") * NS + lax.axis_index("s")
  base = pl.multiple_of(wid * RFULL, 8)
  is_last = wid == (NW - 1)
  r_t = jnp.where(is_last, R_LAST, RFULL)
  iota = lax.broadcasted_iota(jnp.int32, (L,), 0)

  pltpu.sync_copy(idx_hbm, idx_v)

  @pl.when(jnp.logical_not(is_last))
  def _():
    pltpu.sync_copy(cnt_hbm.at[pl.ds(base, RFULL)], cnt_orig)
    pltpu.sync_copy(cnt_hbm.at[pl.ds(base, RFULL)], cnt_new)

  @pl.when(is_last)
  def _():
    pltpu.sync_copy(cnt_hbm.at[pl.ds(base, R_LAST)], cnt_orig.at[pl.ds(0, R_LAST)])
    pltpu.sync_copy(cnt_hbm.at[pl.ds(base, R_LAST)], cnt_new.at[pl.ds(0, R_LAST)])

  def scan_body(i, off):
    v = idx_v[pl.ds(i * L, L)]
    local = v - base
    m = (local >= 0) & (local < r_t)
    plsc.store_compressed(mloc.at[pl.ds(off, L)], local, mask=m)
    plsc.store_compressed(mpos.at[pl.ds(off, L)], iota + i * L, mask=m)
    return off + jnp.sum(m.astype(jnp.int32))

  with jax.named_scope("p1_scan"):
    k_n = lax.fori_loop(0, B // L, scan_body, jnp.int32(0))
  nmc = pl.cdiv(k_n, L)

  def dedup_body(j, _):
    lo = j * L
    vloc = mloc[pl.ds(lo, L)]
    vpos = mpos[pl.ds(lo, L)]
    valid = (iota + lo) < k_n
    later = jnp.zeros((L,), jnp.bool_)
    cnt_e = jnp.zeros((L,), jnp.int32)
    for s in range(1, L):
      v_dn = jnp.take_along_axis(vloc, jnp.minimum(iota + s, L - 1), axis=0)
      v_up = jnp.take_along_axis(vloc, jnp.maximum(iota - s, 0), axis=0)
      ok_dn = (iota + s < L) & ((lo + iota + s) < k_n)
      ok_up = iota - s >= 0
      later = later | (ok_dn & (v_dn == vloc))
      cnt_e = cnt_e + (ok_up & (v_up == vloc)).astype(jnp.int32)
    last = valid & jnp.logical_not(later)
    plsc.store_scatter(last_pos, [vloc], vpos, mask=last)
    plsc.addupdate_scatter(cnt_new, [vloc], cnt_e + 1, mask=last)
    return 0

  with jax.named_scope("p2_dedup"):
    lax.fori_loop(0, nmc, dedup_body, 0)

  def win_body(j, woff):
    lo = j * L
    vloc = mloc[pl.ds(lo, L)]
    vpos = mpos[pl.ds(lo, L)]
    valid = (iota + lo) < k_n
    lp = plsc.load_gather(last_pos, [vloc], mask=valid)
    winner = valid & (lp == vpos)
    plsc.store_compressed(mloc.at[pl.ds(woff, L)], vloc, mask=winner)
    plsc.store_compressed(mpos.at[pl.ds(woff, L)], vpos, mask=winner)
    return woff + jnp.sum(winner.astype(jnp.int32))

  with jax.named_scope("p3_winners"):
    k_w = lax.fori_loop(0, nmc, win_body, jnp.int32(0))
  nwc = pl.cdiv(k_w, L)

  @pl.when(jnp.logical_not(is_last))
  def _():
    pltpu.sync_copy(cnt_new, cntout_hbm.at[pl.ds(base, RFULL)])

  @pl.when(is_last)
  def _():
    pltpu.sync_copy(cnt_new.at[pl.ds(0, R_LAST)], cntout_hbm.at[pl.ds(base, R_LAST)])

  def rescan(c0, cw):
    def chunk_body(j, fill):
      lo = j * L
      vloc = mloc[pl.ds(lo, L)]
      vpos = mpos[pl.ds(lo, L)]
      valid = (iota + lo) < k_w
      m = valid & (vloc + base >= c0) & (vloc + base < c0 + cw)
      plsc.store_compressed(colb.at[pl.ds(fill, L)], vloc + base - c0, mask=m)
      plsc.store_compressed(posb.at[pl.ds(fill, L)], vpos, mask=m)
      c = plsc.load_gather(cnt_orig, [vloc], mask=m)
      isnew = c == 0
      inv = 1.0 / jnp.where(isnew, 1, c).astype(jnp.float32)
      plsc.store_compressed(ab_b.at[pl.ds(fill, L)],
                            jnp.where(isnew, -1.0, inv), mask=m)
      return fill + jnp.sum(m.astype(jnp.int32))

    fill = lax.fori_loop(0, nwc, chunk_body, jnp.int32(0))

    @pl.when(fill > 0)
    def _():
      pos0 = posb[pl.ds(0, L)][0]

      def padp(g, _):
        sl = pl.ds(g * L, L)
        posb[sl] = jnp.where(g * L + iota < fill, posb[sl], pos0)
        return 0

      lax.fori_loop(fill // L, pl.cdiv(fill, VB) * (VB // L), padp, 0)
    return fill

  def patch_all(fill, bufref):
    def batch_body(bat, _):
      with jax.named_scope("u_dma"):
        pltpu.async_copy(valp_hbm.at[posb.at[pl.ds(bat * VB, VB)]],
                         valbuf, sem2).wait()

      def patch_group(gl, _):
        g = bat * (VB // L) + gl
        sl16 = pl.ds(g * L, L)
        valid = (g * L + iota) < fill
        cvec = jnp.where(valid, colb[sl16], 0)
        ab = ab_b[sl16]
        isnew = ab < 0
        avec = jnp.where(isnew, 1.0, ab)
        bvec = jnp.where(isnew, 0.0, ab)

        def patch_q(q, _):
          vs = [valbuf[gl * L + j, pl.ds(q * L, L)] for j in range(L)]
          for s in (1, 2, 4, 8):
            dn = jnp.maximum(iota - s, 0)
            up = jnp.minimum(iota + s, L - 1)
            hi = (iota & s) == 0
            for i in range(L):
              if i & s:
                continue
              a, b = vs[i], vs[i + s]
              vs[i] = jnp.where(hi, a, jnp.take_along_axis(b, dn, axis=0))
              vs[i + s] = jnp.where(hi, jnp.take_along_axis(a, up, axis=0), b)
          for dd in range(L):
            d = q * L + dd
            dvec = jnp.broadcast_to(d, (L,))
            old = plsc.load_gather(bufref, [dvec, cvec], mask=valid)
            newv = avec * vs[dd] + bvec * old
            plsc.store_scatter(bufref, [dvec, cvec], newv, mask=valid)
          return 0

        lax.fori_loop(0, D // L, patch_q, 0)
        return 0

      ngl = jnp.minimum(pl.cdiv(fill - bat * VB, L), VB // L)
      with jax.named_scope("u_patch"):
        lax.fori_loop(0, ngl, patch_group, 0)
      return 0

    lax.fori_loop(0, pl.cdiv(fill, VB), batch_body, 0)

  nb = jnp.where(is_last, 2, RFULL // CW + 1)
  lim = jnp.where(is_last, M - (M % 128) - CW, base + RFULL - CW)

  def full_block(bi, _):
    c0 = pl.multiple_of(jnp.minimum(base + bi * CW, lim), 128)
    cp_in = pltpu.async_copy(memT_hbm.at[:, pl.ds(c0, CW)], sbuf, sem1)
    with jax.named_scope("s_scan"):
      fill = rescan(c0, CW)
    with jax.named_scope("s_in"):
      cp_in.wait()
    with jax.named_scope("s_upd"):
      patch_all(fill, sbuf)
    with jax.named_scope("s_out"):
      pltpu.sync_copy(sbuf, outT_hbm.at[:, pl.ds(c0, CW)])
    return 0

  with jax.named_scope("p4_stream"):
    lax.fori_loop(0, nb, full_block, 0)

  @pl.when(is_last)
  def _():
    c0 = M - (M % 128)
    pltpu.sync_copy(memT_hbm.at[:, pl.ds(c0, M - c0)], edgebuf)
    patch_all(rescan(c0, M - c0), edgebuf)
    pltpu.sync_copy(edgebuf, outT_hbm.at[:, pl.ds(c0, M - c0)])


@functools.cache
def _make_sc_store(interpret=False):
  return pl.kernel(
      _body,
      out_type=(
          jax.ShapeDtypeStruct((D, M), jnp.float32),
          jax.ShapeDtypeStruct((M,), jnp.int32),
      ),
      mesh=_mesh(),
      interpret=interpret,
      compiler_params=pltpu.CompilerParams(needs_layout_passes=False),
      scratch_types=[
          pltpu.VMEM((B,), jnp.int32),
          pltpu.VMEM((B,), jnp.int32),
          pltpu.VMEM((B,), jnp.int32),
          pltpu.VMEM((RFULL,), jnp.int32),
          pltpu.VMEM((RFULL,), jnp.int32),
          pltpu.VMEM((RFULL,), jnp.int32),
          pltpu.VMEM((D, CW), jnp.float32),
          pltpu.VMEM((D, 32), jnp.float32),
          pltpu.VMEM((VB, 2 * D), jnp.float32),
          pltpu.VMEM((LCAP,), jnp.int32),
          pltpu.VMEM((LCAP,), jnp.int32),
          pltpu.VMEM((LCAP,), jnp.float32),
          pltpu.SemaphoreType.DMA,
          pltpu.SemaphoreType.DMA,
      ],
  )


def kernel(mem, counts, val, idx):
  memT = mem.T
  valp = jnp.pad(val, ((0, 0), (0, D)))
  outT, new_counts = _make_sc_store()(memT, counts, valp, idx)
  return outT.T, new_counts


# speedup vs baseline: 1.5121x; 1.3192x over previous
import functools

import jax
import jax.numpy as jnp
from jax import lax
from jax.experimental import pallas as pl
from jax.experimental.pallas import tpu as pltpu
from jax.experimental.pallas import tpu_sc as plsc

M, D, B = 100000, 64, 16384
NC, NS, L = 2, 16, 16
NW = NC * NS
RFULL = 3200
R_LAST = M - RFULL * (NW - 1)
CW = 512
VB = 112
LCAP = -(-CW // VB) * VB + L


def _mesh():
  return plsc.VectorSubcoreMesh(
      core_axis_name="c", subcore_axis_name="s", num_cores=NC, num_subcores=NS
  )


def _body(memT_hbm, cnt_hbm, valp_hbm, idx_hbm, outT_hbm, cntout_hbm,
          idx_v, mloc, mpos, last_pos, cnt_orig, cnt_new,
          sbuf, edgebuf, valbuf, colb, posb, ab_b, sem1, sem2):
  wid = lax.axis_index("c") * NS + lax.axis_index("s")
  base = pl.multiple_of(wid * RFULL, 8)
  is_last = wid == (NW - 1)
  r_t = jnp.where(is_last, R_LAST, RFULL)
  iota = lax.broadcasted_iota(jnp.int32, (L,), 0)

  pltpu.sync_copy(idx_hbm, idx_v)

  @pl.when(jnp.logical_not(is_last))
  def _():
    pltpu.sync_copy(cnt_hbm.at[pl.ds(base, RFULL)], cnt_orig)
    pltpu.sync_copy(cnt_hbm.at[pl.ds(base, RFULL)], cnt_new)

  @pl.when(is_last)
  def _():
    pltpu.sync_copy(cnt_hbm.at[pl.ds(base, R_LAST)], cnt_orig.at[pl.ds(0, R_LAST)])
    pltpu.sync_copy(cnt_hbm.at[pl.ds(base, R_LAST)], cnt_new.at[pl.ds(0, R_LAST)])

  def scan_body(i, off):
    v = idx_v[pl.ds(i * L, L)]
    local = v - base
    m = (local >= 0) & (local < r_t)
    plsc.store_compressed(mloc.at[pl.ds(off, L)], local, mask=m)
    plsc.store_compressed(mpos.at[pl.ds(off, L)], iota + i * L, mask=m)
    return off + jnp.sum(m.astype(jnp.int32))

  with jax.named_scope("p1_scan"):
    k_n = lax.fori_loop(0, B // L, scan_body, jnp.int32(0))
  nmc = pl.cdiv(k_n, L)

  def dedup_body(j, _):
    lo = j * L
    vloc = mloc[pl.ds(lo, L)]
    vpos = mpos[pl.ds(lo, L)]
    valid = (iota + lo) < k_n
    later = jnp.zeros((L,), jnp.bool_)
    cnt_e = jnp.zeros((L,), jnp.int32)
    for s in range(1, L):
      v_dn = jnp.take_along_axis(vloc, jnp.minimum(iota + s, L - 1), axis=0)
      v_up = jnp.take_along_axis(vloc, jnp.maximum(iota - s, 0), axis=0)
      ok_dn = (iota + s < L) & ((lo + iota + s) < k_n)
      ok_up = iota - s >= 0
      later = later | (ok_dn & (v_dn == vloc))
      cnt_e = cnt_e + (ok_up & (v_up == vloc)).astype(jnp.int32)
    last = valid & jnp.logical_not(later)
    plsc.store_scatter(last_pos, [vloc], vpos, mask=last)
    plsc.addupdate_scatter(cnt_new, [vloc], cnt_e + 1, mask=last)
    return 0

  with jax.named_scope("p2_dedup"):
    lax.fori_loop(0, nmc, dedup_body, 0)

  def win_body(j, woff):
    lo = j * L
    vloc = mloc[pl.ds(lo, L)]
    vpos = mpos[pl.ds(lo, L)]
    valid = (iota + lo) < k_n
    lp = plsc.load_gather(last_pos, [vloc], mask=valid)
    winner = valid & (lp == vpos)
    plsc.store_compressed(mloc.at[pl.ds(woff, L)], vloc, mask=winner)
    plsc.store_compressed(mpos.at[pl.ds(woff, L)], vpos, mask=winner)
    return woff + jnp.sum(winner.astype(jnp.int32))

  with jax.named_scope("p3_winners"):
    k_w = lax.fori_loop(0, nmc, win_body, jnp.int32(0))
  nwc = pl.cdiv(k_w, L)

  @pl.when(jnp.logical_not(is_last))
  def _():
    pltpu.sync_copy(cnt_new, cntout_hbm.at[pl.ds(base, RFULL)])

  @pl.when(is_last)
  def _():
    pltpu.sync_copy(cnt_new.at[pl.ds(0, R_LAST)], cntout_hbm.at[pl.ds(base, R_LAST)])

  def rescan(c0, cw):
    def chunk_body(j, fill):
      lo = j * L
      vloc = mloc[pl.ds(lo, L)]
      vpos = mpos[pl.ds(lo, L)]
      valid = (iota + lo) < k_w
      m = valid & (vloc + base >= c0) & (vloc + base < c0 + cw)
      plsc.store_compressed(colb.at[pl.ds(fill, L)], vloc + base - c0, mask=m)
      plsc.store_compressed(posb.at[pl.ds(fill, L)], vpos, mask=m)
      c = plsc.load_gather(cnt_orig, [vloc], mask=m)
      isnew = c == 0
      inv = 1.0 / jnp.where(isnew, 1, c).astype(jnp.float32)
      plsc.store_compressed(ab_b.at[pl.ds(fill, L)],
                            jnp.where(isnew, -1.0, inv), mask=m)
      return fill + jnp.sum(m.astype(jnp.int32))

    fill = lax.fori_loop(0, nwc, chunk_body, jnp.int32(0))

    @pl.when(fill > 0)
    def _():
      pos0 = posb[pl.ds(0, L)][0]

      def padp(g, _):
        sl = pl.ds(g * L, L)
        posb[sl] = jnp.where(g * L + iota < fill, posb[sl], pos0)
        return 0

      lax.fori_loop(fill // L, pl.cdiv(fill, VB) * (VB // L), padp, 0)
    return fill

  def fire_val0(fill):
    @pl.when(fill > 0)
    def _():
      pltpu.async_copy(valp_hbm.at[posb.at[pl.ds(0, VB)]], valbuf, sem2)

  def patch_all(fill, bufref):
    def patch_batch(bat, ngl):
      def patch_group(gl, _):
        g = bat * (VB // L) + gl
        sl16 = pl.ds(g * L, L)
        valid = (g * L + iota) < fill
        cvec = jnp.where(valid, colb[sl16], 0)
        ab = ab_b[sl16]
        isnew = ab < 0
        avec = jnp.where(isnew, 1.0, ab)
        bvec = jnp.where(isnew, 0.0, ab)

        def patch_q(q, _):
          vs = [valbuf[gl * L + j, pl.ds(q * L, L)] for j in range(L)]
          for s in (1, 2, 4, 8):
            dn = jnp.maximum(iota - s, 0)
            up = jnp.minimum(iota + s, L - 1)
            hi = (iota & s) == 0
            for i in range(L):
              if i & s:
                continue
              a, b = vs[i], vs[i + s]
              vs[i] = jnp.where(hi, a, jnp.take_along_axis(b, dn, axis=0))
              vs[i + s] = jnp.where(hi, jnp.take_along_axis(a, up, axis=0), b)
          for dd in range(L):
            d = q * L + dd
            dvec = jnp.broadcast_to(d, (L,))
            old = plsc.load_gather(bufref, [dvec, cvec], mask=valid)
            newv = avec * vs[dd] + bvec * old
            plsc.store_scatter(bufref, [dvec, cvec], newv, mask=valid)
          return 0

        lax.fori_loop(0, D // L, patch_q, 0)
        return 0

      lax.fori_loop(0, ngl, patch_group, 0)

    @pl.when(fill > 0)
    def _():
      with jax.named_scope("u_dma"):
        pltpu.make_async_copy(valp_hbm.at[posb.at[pl.ds(0, VB)]],
                              valbuf, sem2).wait()
      with jax.named_scope("u_patch"):
        patch_batch(0, jnp.minimum(pl.cdiv(fill, L), VB // L))

    def batch_body(bat, _):
      with jax.named_scope("u_dma"):
        pltpu.async_copy(valp_hbm.at[posb.at[pl.ds(bat * VB, VB)]],
                         valbuf, sem2).wait()
      with jax.named_scope("u_patch"):
        patch_batch(bat, jnp.minimum(pl.cdiv(fill - bat * VB, L), VB // L))
      return 0

    lax.fori_loop(1, pl.cdiv(fill, VB), batch_body, 0)

  nb = jnp.where(is_last, 2, RFULL // CW + 1)
  lim = jnp.where(is_last, M - (M % 128) - CW, base + RFULL - CW)

  def full_block(bi, _):
    c0 = pl.multiple_of(jnp.minimum(base + bi * CW, lim), 128)
    cp_in = pltpu.async_copy(memT_hbm.at[:, pl.ds(c0, CW)], sbuf, sem1)
    with jax.named_scope("s_scan"):
      fill = rescan(c0, CW)
    fire_val0(fill)
    with jax.named_scope("s_in"):
      cp_in.wait()
    with jax.named_scope("s_upd"):
      patch_all(fill, sbuf)
    with jax.named_scope("s_out"):
      pltpu.sync_copy(sbuf, outT_hbm.at[:, pl.ds(c0, CW)])
    return 0

  with jax.named_scope("p4_stream"):
    lax.fori_loop(0, nb, full_block, 0)

  @pl.when(is_last)
  def _():
    c0 = M - (M % 128)
    pltpu.sync_copy(memT_hbm.at[:, pl.ds(c0, M - c0)], edgebuf)
    fe = rescan(c0, M - c0)
    fire_val0(fe)
    patch_all(fe, edgebuf)
    pltpu.sync_copy(edgebuf, outT_hbm.at[:, pl.ds(c0, M - c0)])


@functools.cache
def _make_sc_store(interpret=False):
  return pl.kernel(
      _body,
      out_type=(
          jax.ShapeDtypeStruct((D, M), jnp.float32),
          jax.ShapeDtypeStruct((M,), jnp.int32),
      ),
      mesh=_mesh(),
      interpret=interpret,
      compiler_params=pltpu.CompilerParams(needs_layout_passes=False),
      scratch_types=[
          pltpu.VMEM((B,), jnp.int32),
          pltpu.VMEM((B,), jnp.int32),
          pltpu.VMEM((B,), jnp.int32),
          pltpu.VMEM((RFULL,), jnp.int32),
          pltpu.VMEM((RFULL,), jnp.int32),
          pltpu.VMEM((RFULL,), jnp.int32),
          pltpu.VMEM((D, CW), jnp.float32),
          pltpu.VMEM((D, 32), jnp.float32),
          pltpu.VMEM((VB, 2 * D), jnp.float32),
          pltpu.VMEM((LCAP,), jnp.int32),
          pltpu.VMEM((LCAP,), jnp.int32),
          pltpu.VMEM((LCAP,), jnp.float32),
          pltpu.SemaphoreType.DMA,
          pltpu.SemaphoreType.DMA,
      ],
  )


def kernel(mem, counts, val, idx):
  memT = mem.T
  valp = jnp.pad(val, ((0, 0), (0, D)))
  outT, new_counts = _make_sc_store()(memT, counts, valp, idx)
  return outT.T, new_counts
